# Initial kernel scaffold; baseline (speedup 1.0000x reference)
import jax
import jax.numpy as jnp
from jax.experimental import pallas as pl


def kernel(feats, coords, pre_volume):
    raise NotImplementedError("write your pallas kernel here")


# speedup vs baseline: 4.2290x; 4.2290x over previous
import functools

import numpy as np
import jax
import jax.numpy as jnp
from jax import lax
from jax.experimental import pallas as pl
from jax.experimental.pallas import tpu as pltpu
from jax.experimental.pallas import tpu_sc as plsc

_VOL = 128
_NV = _VOL * _VOL * _VOL
_C = 16
_NPTS = 500000
_NC = 2
_NS = 16
_NW = _NC * _NS
_W = _NV // _NW
_WROWS = _W // 128
_CH = 4000
_PADMASK = (1 << 18) - 1


def _ids_body(x_ref, y_ref, z_ref, o_ref):
    o_ref[...] = (x_ref[...] * (_VOL * _VOL) + y_ref[...] * _VOL
                  + z_ref[...])


def _flat_ids(coords):
    x = coords[:, 0].reshape(125, 1, _CH)
    y = coords[:, 1].reshape(125, 1, _CH)
    z = coords[:, 2].reshape(125, 1, _CH)
    spec = pl.BlockSpec((1, 1, _CH), lambda i: (i, 0, 0))
    ids = pl.pallas_call(
        _ids_body,
        grid=(125,),
        in_specs=[spec, spec, spec],
        out_specs=spec,
        out_shape=jax.ShapeDtypeStruct((125, 1, _CH), jnp.int32),
    )(x, y, z)
    return ids.reshape(_NPTS)


def _sc_body(ids_hbm, feats_hbm, idxvol_hbm, g_hbm, lbuf, idbuf, rowbuf,
             sem):
    wid = lax.axis_index("s") * _NC + lax.axis_index("c")
    lo = wid * _W
    lane = lax.iota(jnp.int32, 16)

    neg1 = jnp.full((16,), -1, jnp.int32)

    def init_body(i, carry):
        lbuf[pl.ds(pl.multiple_of(i * 16, 8), 16)] = neg1
        return carry

    lax.fori_loop(0, _W // 16, init_body, 0)

    def chunk_body(k, carry):
        pltpu.sync_copy(
            ids_hbm.at[pl.ds(pl.multiple_of(k * _CH, 8), _CH)], idbuf)
        base = k * _CH

        def vec_body(j, c2):
            v = idbuf[pl.ds(pl.multiple_of(j * 16, 8), 16)]
            u = v - lo
            m = (u >= 0) & (u < _W)
            us = jnp.where(m, u, 0)
            val = base + j * 16 + lane
            plsc.store_scatter(lbuf, [us], val, mask=m)
            return c2

        lax.fori_loop(0, _CH // 16, vec_body, 0)
        return carry

    lax.fori_loop(0, _NPTS // _CH, chunk_body, 0)

    pltpu.sync_copy(lbuf, idxvol_hbm.at[pl.ds(pl.multiple_of(lo, 8), _W)])

    def clamp_body(i, carry):
        off = pl.multiple_of(i * 16, 8)
        xv = lbuf[pl.ds(off, 16)]
        pad = (lo + i * 16 + lane) & _PADMASK
        lbuf[pl.ds(off, 16)] = jnp.where(xv >= 0, xv, pad)
        return carry

    lax.fori_loop(0, _W // 16, clamp_body, 0)

    def gather_body(j, carry):
        idxs = lbuf.at[pl.ds(pl.multiple_of(j * 128, 8), 128)]
        pltpu.async_copy(feats_hbm.at[idxs], rowbuf, sem).wait()
        pltpu.sync_copy(
            rowbuf,
            g_hbm.at[pl.ds(pl.multiple_of(lo + j * 128, 8), 128)])
        return carry

    lax.fori_loop(0, _WROWS, gather_body, 0)


@functools.cache
def _sc_scatter_gather():
    return pl.kernel(
        _sc_body,
        out_type=(
            jax.ShapeDtypeStruct((_NV,), jnp.int32),
            jax.ShapeDtypeStruct((_NV, _C), jnp.float32),
        ),
        mesh=plsc.VectorSubcoreMesh(core_axis_name="c",
                                    subcore_axis_name="s",
                                    num_cores=_NC, num_subcores=_NS),
        compiler_params=pltpu.CompilerParams(needs_layout_passes=False,
                                             use_tc_tiling_on_sc=False),
        scratch_types=[
            pltpu.VMEM((_W,), jnp.int32),
            pltpu.VMEM((_CH,), jnp.int32),
            pltpu.VMEM((128, _C), jnp.float32),
            pltpu.SemaphoreType.DMA,
        ],
    )


def _upsample_matrix():
    xs = np.clip((np.arange(128, dtype=np.float64) + 0.5) / 2.0 - 0.5,
                 0.0, 63.0)
    x0 = np.floor(xs).astype(np.int64)
    x1 = np.minimum(x0 + 1, 63)
    w = xs - x0
    u = np.zeros((128, 64), dtype=np.float32)
    u[np.arange(128), x0] += (1.0 - w).astype(np.float32)
    u[np.arange(128), x1] += w.astype(np.float32)
    return u


def _asm_body(idx_ref, g_ref, pre_ref, uy_ref, uyt_ref, dense_ref,
              mask_ref):
    step = pl.program_id(0)
    px = step >> 2
    q = step & 3

    idx = idx_ref[0, 0, :]
    occ = idx >= 0
    gblk = g_ref[0]

    eye = (lax.broadcasted_iota(jnp.int32, (_C, _C), 0)
           == lax.broadcasted_iota(jnp.int32, (_C, _C), 1)
           ).astype(jnp.float32)
    gt = lax.dot_general(eye, gblk, (((1,), (1,)), ((), ())),
                         preferred_element_type=jnp.float32)

    xf = jnp.clip((px.astype(jnp.float32) + 0.5) / 2.0 - 0.5, 0.0, 63.0)
    x0 = jnp.floor(xf).astype(jnp.int32)
    x1 = jnp.minimum(x0 + 1, 63)
    w = xf - x0.astype(jnp.float32)
    pa = pre_ref[pl.ds(x0, 1)][0]
    pb = pre_ref[pl.ds(x1, 1)][0]
    plane = pa * (1.0 - w) + pb * w
    uyq = uy_ref[pl.ds(q * 32, 32), :]
    dq = jnp.dot(jnp.dot(uyq, plane, preferred_element_type=jnp.float32),
                 uyt_ref[...],
                 preferred_element_type=jnp.float32)
    dflat = dq.reshape(1, 4096)

    ch = lax.broadcasted_iota(jnp.int32, (_C, 4096), 0)
    bg = jnp.where(ch == 0, jnp.broadcast_to(dflat, (_C, 4096)), 0.0)
    out = jnp.where(occ[None, :], gt, bg)
    dense_ref[...] = out
    mask_ref[0, 0, :] = occ.astype(jnp.float32)


def _assemble(idxvol, g, pre, uy, uyt):
    nblk = _NV // 4096
    dense, mask = pl.pallas_call(
        _asm_body,
        grid=(nblk,),
        in_specs=[
            pl.BlockSpec((1, 1, 4096), lambda i: (i, 0, 0)),
            pl.BlockSpec((1, 4096, _C), lambda i: (i, 0, 0)),
            pl.BlockSpec((64, 64, 64), lambda i: (0, 0, 0)),
            pl.BlockSpec((128, 64), lambda i: (0, 0)),
            pl.BlockSpec((64, 128), lambda i: (0, 0)),
        ],
        out_specs=[
            pl.BlockSpec((_C, 4096), lambda i: (0, i)),
            pl.BlockSpec((1, 1, 4096), lambda i: (i, 0, 0)),
        ],
        out_shape=[
            jax.ShapeDtypeStruct((_C, _NV), jnp.float32),
            jax.ShapeDtypeStruct((nblk, 1, 4096), jnp.float32),
        ],
    )(idxvol.reshape(nblk, 1, 4096), g.reshape(nblk, 4096, _C), pre, uy,
      uyt)
    return dense, mask


def kernel(feats, coords, pre_volume):
    ids = _flat_ids(coords.astype(jnp.int32))
    idxvol, g = _sc_scatter_gather()(ids, feats)
    uy = jnp.asarray(_upsample_matrix())
    dense, mask = _assemble(idxvol, g,
                            pre_volume.reshape(64, 64, 64), uy, uy.T)
    dense_volume = dense.reshape(1, _C, _VOL, _VOL, _VOL)
    mask_volume = mask.reshape(1, 1, _VOL, _VOL, _VOL)
    return dense_volume, mask_volume


# speedup vs baseline: 5.2248x; 1.2355x over previous
import functools

import numpy as np
import jax
import jax.numpy as jnp
from jax import lax
from jax.experimental import pallas as pl
from jax.experimental.pallas import tpu as pltpu
from jax.experimental.pallas import tpu_sc as plsc

_VOL = 128
_NV = _VOL * _VOL * _VOL
_C = 16
_NPTS = 500000
_NC = 2
_NS = 16
_NW = _NC * _NS
_W = _NV // _NW
_WROWS = _W // 128
_CH = 4000
_PADMASK = (1 << 18) - 1


def _ids_body(x_ref, y_ref, z_ref, o_ref):
    o_ref[...] = (x_ref[...] * (_VOL * _VOL) + y_ref[...] * _VOL
                  + z_ref[...])


def _flat_ids(coords):
    x = coords[:, 0].reshape(125, 1, _CH)
    y = coords[:, 1].reshape(125, 1, _CH)
    z = coords[:, 2].reshape(125, 1, _CH)
    spec = pl.BlockSpec((1, 1, _CH), lambda i: (i, 0, 0))
    ids = pl.pallas_call(
        _ids_body,
        grid=(125,),
        in_specs=[spec, spec, spec],
        out_specs=spec,
        out_shape=jax.ShapeDtypeStruct((125, 1, _CH), jnp.int32),
    )(x, y, z)
    return ids.reshape(_NPTS)


_GCH = 1024
_NGCH = _W // _GCH
_GSTR = _GCH // 128


def _sc_body(ids_hbm, feats_hbm, idxvol_hbm, g_hbm, lbuf, idbuf, gbuf,
             semi, semg):
    wid = lax.axis_index("s") * _NC + lax.axis_index("c")
    lo = wid * _W
    lane = lax.iota(jnp.int32, 16)
    neg1 = jnp.full((16,), -1, jnp.int32)

    def init_body(i, carry):
        for t in range(8):
            lbuf[pl.ds(pl.multiple_of((i * 8 + t) * 16, 8), 16)] = neg1
        return carry

    lax.fori_loop(0, _W // 128, init_body, 0)

    nchunk = _NPTS // _CH

    def id_copy(k, b):
        return pltpu.make_async_copy(
            ids_hbm.at[pl.ds(pl.multiple_of(k * _CH, 8), _CH)],
            idbuf.at[b], semi)

    id_copy(0, 0).start()

    def chunk_body(k, carry):
        b = k & 1
        id_copy(k, b).wait()
        pl.when(k < nchunk - 1)(
            lambda: id_copy(k + 1, b ^ 1).start())
        base = k * _CH

        def vec_body(j, c2):
            for t in range(10):
                off = pl.multiple_of((j * 10 + t) * 16, 8)
                v = idbuf[b, pl.ds(off, 16)]
                u = v - lo
                m = (u >= 0) & (u < _W)
                us = jnp.where(m, u, 0)
                val = base + (j * 10 + t) * 16 + lane
                plsc.store_scatter(lbuf, [us], val, mask=m)
            return c2

        lax.fori_loop(0, _CH // 160, vec_body, 0)
        return carry

    lax.fori_loop(0, nchunk, chunk_body, 0)

    pltpu.sync_copy(lbuf, idxvol_hbm.at[pl.ds(pl.multiple_of(lo, 8), _W)])

    def clamp_body(i, carry):
        for t in range(8):
            off = pl.multiple_of((i * 8 + t) * 16, 8)
            xv = lbuf[pl.ds(off, 16)]
            pad = (lo + (i * 8 + t) * 16 + lane) & _PADMASK
            lbuf[pl.ds(off, 16)] = jnp.where(xv >= 0, xv, pad)
        return carry

    lax.fori_loop(0, _W // 128, clamp_body, 0)

    def fire(c, b):
        for j in range(_GSTR):
            idxs = lbuf.at[pl.ds(pl.multiple_of(c * _GCH + j * 128, 8),
                                 128)]
            pltpu.make_async_copy(
                feats_hbm.at[idxs],
                gbuf.at[b, pl.ds(j * 128, 128)], semg).start()

    def drain(b):
        for j in range(_GSTR):
            idxs = lbuf.at[pl.ds(j * 128, 128)]
            pltpu.make_async_copy(
                feats_hbm.at[idxs],
                gbuf.at[b, pl.ds(j * 128, 128)], semg).wait()

    fire(0, 0)

    def gather_body(c, carry):
        b = c & 1
        pl.when(c < _NGCH - 1)(lambda: fire(c + 1, b ^ 1))
        drain(b)
        pltpu.sync_copy(
            gbuf.at[b],
            g_hbm.at[pl.ds(pl.multiple_of(lo + c * _GCH, 8), _GCH)])
        return carry

    lax.fori_loop(0, _NGCH, gather_body, 0)


@functools.cache
def _sc_scatter_gather():
    return pl.kernel(
        _sc_body,
        out_type=(
            jax.ShapeDtypeStruct((_NV,), jnp.int32),
            jax.ShapeDtypeStruct((_NV, _C), jnp.float32),
        ),
        mesh=plsc.VectorSubcoreMesh(core_axis_name="c",
                                    subcore_axis_name="s",
                                    num_cores=_NC, num_subcores=_NS),
        compiler_params=pltpu.CompilerParams(needs_layout_passes=False,
                                             use_tc_tiling_on_sc=False),
        scratch_types=[
            pltpu.VMEM((_W,), jnp.int32),
            pltpu.VMEM((2, _CH), jnp.int32),
            pltpu.VMEM((2, _GCH, _C), jnp.float32),
            pltpu.SemaphoreType.DMA,
            pltpu.SemaphoreType.DMA,
        ],
    )


def _upsample_matrix():
    xs = np.clip((np.arange(128, dtype=np.float64) + 0.5) / 2.0 - 0.5,
                 0.0, 63.0)
    x0 = np.floor(xs).astype(np.int64)
    x1 = np.minimum(x0 + 1, 63)
    w = xs - x0
    u = np.zeros((128, 64), dtype=np.float32)
    u[np.arange(128), x0] += (1.0 - w).astype(np.float32)
    u[np.arange(128), x1] += w.astype(np.float32)
    return u


def _asm_body(idx_ref, g_ref, pre_ref, uy_ref, uyt_ref, dense_ref,
              mask_ref):
    step = pl.program_id(0)
    px = step >> 2
    q = step & 3

    idx = idx_ref[0, 0, :]
    occ = idx >= 0
    gblk = g_ref[0]

    eye = (lax.broadcasted_iota(jnp.int32, (_C, _C), 0)
           == lax.broadcasted_iota(jnp.int32, (_C, _C), 1)
           ).astype(jnp.float32)
    gt = lax.dot_general(eye, gblk, (((1,), (1,)), ((), ())),
                         preferred_element_type=jnp.float32)

    xf = jnp.clip((px.astype(jnp.float32) + 0.5) / 2.0 - 0.5, 0.0, 63.0)
    x0 = jnp.floor(xf).astype(jnp.int32)
    x1 = jnp.minimum(x0 + 1, 63)
    w = xf - x0.astype(jnp.float32)
    pa = pre_ref[pl.ds(x0, 1)][0]
    pb = pre_ref[pl.ds(x1, 1)][0]
    plane = pa * (1.0 - w) + pb * w
    uyq = uy_ref[pl.ds(q * 32, 32), :]
    dq = jnp.dot(jnp.dot(uyq, plane, preferred_element_type=jnp.float32),
                 uyt_ref[...],
                 preferred_element_type=jnp.float32)
    dflat = dq.reshape(1, 4096)

    ch = lax.broadcasted_iota(jnp.int32, (_C, 4096), 0)
    bg = jnp.where(ch == 0, jnp.broadcast_to(dflat, (_C, 4096)), 0.0)
    out = jnp.where(occ[None, :], gt, bg)
    dense_ref[...] = out
    mask_ref[0, 0, :] = occ.astype(jnp.float32)


def _assemble(idxvol, g, pre, uy, uyt):
    nblk = _NV // 4096
    dense, mask = pl.pallas_call(
        _asm_body,
        grid=(nblk,),
        in_specs=[
            pl.BlockSpec((1, 1, 4096), lambda i: (i, 0, 0)),
            pl.BlockSpec((1, 4096, _C), lambda i: (i, 0, 0)),
            pl.BlockSpec((64, 64, 64), lambda i: (0, 0, 0)),
            pl.BlockSpec((128, 64), lambda i: (0, 0)),
            pl.BlockSpec((64, 128), lambda i: (0, 0)),
        ],
        out_specs=[
            pl.BlockSpec((_C, 4096), lambda i: (0, i)),
            pl.BlockSpec((1, 1, 4096), lambda i: (i, 0, 0)),
        ],
        out_shape=[
            jax.ShapeDtypeStruct((_C, _NV), jnp.float32),
            jax.ShapeDtypeStruct((nblk, 1, 4096), jnp.float32),
        ],
    )(idxvol.reshape(nblk, 1, 4096), g.reshape(nblk, 4096, _C), pre, uy,
      uyt)
    return dense, mask


def kernel(feats, coords, pre_volume):
    ids = _flat_ids(coords.astype(jnp.int32))
    idxvol, g = _sc_scatter_gather()(ids, feats)
    uy = jnp.asarray(_upsample_matrix())
    dense, mask = _assemble(idxvol, g,
                            pre_volume.reshape(64, 64, 64), uy, uy.T)
    dense_volume = dense.reshape(1, _C, _VOL, _VOL, _VOL)
    mask_volume = mask.reshape(1, 1, _VOL, _VOL, _VOL)
    return dense_volume, mask_volume


# speedup vs baseline: 8.5168x; 1.6301x over previous
import functools

import numpy as np
import jax
import jax.numpy as jnp
from jax import lax
from jax.experimental import pallas as pl
from jax.experimental.pallas import tpu as pltpu
from jax.experimental.pallas import tpu_sc as plsc

_VOL = 128
_NV = _VOL * _VOL * _VOL
_C = 16
_NPTS = 500000
_NC = 2
_NS = 16
_NW = _NC * _NS
_W = _NV // _NW
_WROWS = _W // 128
_CH = 4000
_PADMASK = (1 << 18) - 1


def _ids_body(x_ref, y_ref, z_ref, o_ref):
    o_ref[...] = (x_ref[...] * (_VOL * _VOL) + y_ref[...] * _VOL
                  + z_ref[...])


def _flat_ids(coords):
    x = coords[:, 0].reshape(125, 1, _CH)
    y = coords[:, 1].reshape(125, 1, _CH)
    z = coords[:, 2].reshape(125, 1, _CH)
    spec = pl.BlockSpec((1, 1, _CH), lambda i: (i, 0, 0))
    ids = pl.pallas_call(
        _ids_body,
        grid=(125,),
        in_specs=[spec, spec, spec],
        out_specs=spec,
        out_shape=jax.ShapeDtypeStruct((125, 1, _CH), jnp.int32),
    )(x, y, z)
    return ids.reshape(_NPTS)


_GCH = 1024
_NGCH = _W // _GCH
_GSTR = _GCH // 128


def _sc_body(ids_hbm, feats_hbm, dens_hbm, dense_hbm, mask_hbm,
             lbuf, idbuf, gbuf, tbuf, mkbuf, dnbuf, semi, semg0, semg1,
             semw):
    wid = lax.axis_index("s") * _NC + lax.axis_index("c")
    lo = wid * _W
    lane = lax.iota(jnp.int32, 16)
    neg1 = jnp.full((16,), -1, jnp.int32)

    def init_body(i, carry):
        for t in range(8):
            lbuf[pl.ds(pl.multiple_of((i * 8 + t) * 16, 8), 16)] = neg1
        return carry

    lax.fori_loop(0, _W // 128, init_body, 0)

    nchunk = _NPTS // _CH

    def id_copy(k, b):
        return pltpu.make_async_copy(
            ids_hbm.at[pl.ds(pl.multiple_of(k * _CH, 8), _CH)],
            idbuf.at[b], semi)

    id_copy(0, 0).start()

    def chunk_body(k, carry):
        b = k & 1
        id_copy(k, b).wait()
        pl.when(k < nchunk - 1)(
            lambda: id_copy(k + 1, b ^ 1).start())
        base = k * _CH

        def vec_body(j, c2):
            for t in range(10):
                off = pl.multiple_of((j * 10 + t) * 16, 8)
                v = idbuf[b, pl.ds(off, 16)]
                u = v - lo
                m = (u >= 0) & (u < _W)
                us = jnp.where(m, u, 0)
                val = base + (j * 10 + t) * 16 + lane
                plsc.store_scatter(lbuf, [us], val, mask=m)
            return c2

        lax.fori_loop(0, _CH // 160, vec_body, 0)
        return carry

    lax.fori_loop(0, nchunk, chunk_body, 0)

    def prep(c, b):
        base = c * _GCH

        def pbody(j, carry):
            for t in range(4):
                jj = j * 4 + t
                off = pl.multiple_of(base + jj * 16, 8)
                xv = lbuf[pl.ds(off, 16)]
                m = xv >= 0
                pad = (lo + base + jj * 16 + lane) & _PADMASK
                lbuf[pl.ds(off, 16)] = jnp.where(m, xv, pad)
                mkbuf[b, pl.ds(pl.multiple_of(jj * 16, 8), 16)] = (
                    jnp.where(m, 1.0, 0.0))
            return carry

        lax.fori_loop(0, _GCH // 64, pbody, 0)

    def fire_g(c, b, sg):
        for j in range(_GSTR):
            idxs = lbuf.at[pl.ds(pl.multiple_of(c * _GCH + j * 128, 8),
                                 128)]
            pltpu.make_async_copy(
                feats_hbm.at[idxs],
                gbuf.at[pl.ds(b * _GCH + j * 128, 128)], sg).start()
        pltpu.make_async_copy(
            dens_hbm.at[pl.ds(pl.multiple_of(lo + c * _GCH, 8), _GCH)],
            dnbuf.at[b], sg).start()

    def drain_g(b, sg):
        for j in range(_GSTR):
            idxs = lbuf.at[pl.ds(j * 128, 128)]
            pltpu.make_async_copy(
                feats_hbm.at[idxs],
                gbuf.at[pl.ds(b * _GCH + j * 128, 128)], sg).wait()
        pltpu.make_async_copy(
            dens_hbm.at[pl.ds(0, _GCH)], dnbuf.at[b], sg).wait()

    def xpose(b):
        def jbody(j, carry):
            moff = pl.multiple_of(j * 16, 8)
            mf = mkbuf[b, pl.ds(moff, 16)]
            dn = dnbuf[b, pl.ds(moff, 16)]
            bgz = dn * (1.0 - mf)
            rowv = b * _GCH + j * 16 + lane
            for ch in range(_C):
                cv = jnp.full((16,), ch, jnp.int32)
                val = plsc.load_gather(gbuf, [rowv, cv])
                out = val * mf
                if ch == 0:
                    out = out + bgz
                tbuf[ch, pl.ds(moff, 16)] = out
            return carry

        lax.fori_loop(0, _GCH // 16, jbody, 0)

    def fire_w(c, b):
        dst = pl.ds(pl.multiple_of(lo + c * _GCH, 8), _GCH)
        for ch in range(_C):
            pltpu.make_async_copy(
                tbuf.at[ch], dense_hbm.at[ch, dst], semw).start()
        pltpu.make_async_copy(mkbuf.at[b], mask_hbm.at[dst],
                              semw).start()

    def drain_w():
        dst = pl.ds(0, _GCH)
        for ch in range(_C):
            pltpu.make_async_copy(
                tbuf.at[ch], dense_hbm.at[ch, dst], semw).wait()
        pltpu.make_async_copy(mkbuf.at[0], mask_hbm.at[dst],
                              semw).wait()

    prep(0, 0)
    fire_g(0, 0, semg0)

    def pair_body(p, carry):
        for b in (0, 1):
            c = p * 2 + b
            sg_cur = semg0 if b == 0 else semg1
            sg_nxt = semg1 if b == 0 else semg0

            def donext(c=c, b=b, sg=sg_nxt):
                prep(c + 1, b ^ 1)
                fire_g(c + 1, b ^ 1, sg)

            pl.when(c >= 1)(drain_w)
            pl.when(c < _NGCH - 1)(donext)
            drain_g(b, sg_cur)
            xpose(b)
            fire_w(c, b)
        return carry

    lax.fori_loop(0, _NGCH // 2, pair_body, 0)
    drain_w()


@functools.cache
def _sc_main():
    return pl.kernel(
        _sc_body,
        out_type=(
            jax.ShapeDtypeStruct((_C, _NV), jnp.float32),
            jax.ShapeDtypeStruct((_NV,), jnp.float32),
        ),
        mesh=plsc.VectorSubcoreMesh(core_axis_name="c",
                                    subcore_axis_name="s",
                                    num_cores=_NC, num_subcores=_NS),
        compiler_params=pltpu.CompilerParams(needs_layout_passes=False,
                                             use_tc_tiling_on_sc=False),
        scratch_types=[
            pltpu.VMEM((_W,), jnp.int32),
            pltpu.VMEM((2, _CH), jnp.int32),
            pltpu.VMEM((2 * _GCH, _C), jnp.float32),
            pltpu.VMEM((_C, _GCH), jnp.float32),
            pltpu.VMEM((2, _GCH), jnp.float32),
            pltpu.VMEM((2, _GCH), jnp.float32),
            pltpu.SemaphoreType.DMA,
            pltpu.SemaphoreType.DMA,
            pltpu.SemaphoreType.DMA,
            pltpu.SemaphoreType.DMA,
        ],
    )


def _upsample_matrix():
    xs = np.clip((np.arange(128, dtype=np.float64) + 0.5) / 2.0 - 0.5,
                 0.0, 63.0)
    x0 = np.floor(xs).astype(np.int64)
    x1 = np.minimum(x0 + 1, 63)
    w = xs - x0
    u = np.zeros((128, 64), dtype=np.float32)
    u[np.arange(128), x0] += (1.0 - w).astype(np.float32)
    u[np.arange(128), x1] += w.astype(np.float32)
    return u


def _ups_body(pre_ref, uy_ref, uyt_ref, o_ref):
    px = pl.program_id(0)
    xf = jnp.clip((px.astype(jnp.float32) + 0.5) / 2.0 - 0.5, 0.0, 63.0)
    x0 = jnp.floor(xf).astype(jnp.int32)
    x1 = jnp.minimum(x0 + 1, 63)
    w = xf - x0.astype(jnp.float32)
    pa = pre_ref[pl.ds(x0, 1)][0]
    pb = pre_ref[pl.ds(x1, 1)][0]
    plane = pa * (1.0 - w) + pb * w
    d = jnp.dot(jnp.dot(uy_ref[...], plane,
                        preferred_element_type=jnp.float32),
                uyt_ref[...],
                preferred_element_type=jnp.float32)
    o_ref[0, 0, :] = d.reshape(_VOL * _VOL)


def _upsample(pre, uy, uyt):
    dens = pl.pallas_call(
        _ups_body,
        grid=(_VOL,),
        in_specs=[
            pl.BlockSpec((64, 64, 64), lambda i: (0, 0, 0)),
            pl.BlockSpec((128, 64), lambda i: (0, 0)),
            pl.BlockSpec((64, 128), lambda i: (0, 0)),
        ],
        out_specs=pl.BlockSpec((1, 1, _VOL * _VOL), lambda i: (i, 0, 0)),
        out_shape=jax.ShapeDtypeStruct((_VOL, 1, _VOL * _VOL),
                                       jnp.float32),
    )(pre, uy, uyt)
    return dens.reshape(_NV)


def kernel(feats, coords, pre_volume):
    ids = _flat_ids(coords.astype(jnp.int32))
    uy = jnp.asarray(_upsample_matrix())
    dens = _upsample(pre_volume.reshape(64, 64, 64), uy, uy.T)
    dense, mask = _sc_main()(ids, feats, dens)
    dense_volume = dense.reshape(1, _C, _VOL, _VOL, _VOL)
    mask_volume = mask.reshape(1, 1, _VOL, _VOL, _VOL)
    return dense_volume, mask_volume


# speedup vs baseline: 15.2072x; 1.7856x over previous
import functools

import numpy as np
import jax
import jax.numpy as jnp
from jax import lax
from jax.experimental import pallas as pl
from jax.experimental.pallas import tpu as pltpu
from jax.experimental.pallas import tpu_sc as plsc

_VOL = 128
_NV = _VOL * _VOL * _VOL
_C = 16
_NPTS = 500000
_NC = 2
_NS = 16
_NW = _NC * _NS
_W = _NV // _NW
_WROWS = _W // 128
_CH = 4000
_PADMASK = (1 << 18) - 1


def _ids_body(x_ref, y_ref, z_ref, o_ref):
    o_ref[...] = (x_ref[...] * (_VOL * _VOL) + y_ref[...] * _VOL
                  + z_ref[...])


def _flat_ids(coords):
    x = coords[:, 0].reshape(125, 1, _CH)
    y = coords[:, 1].reshape(125, 1, _CH)
    z = coords[:, 2].reshape(125, 1, _CH)
    spec = pl.BlockSpec((1, 1, _CH), lambda i: (i, 0, 0))
    ids = pl.pallas_call(
        _ids_body,
        grid=(125,),
        in_specs=[spec, spec, spec],
        out_specs=spec,
        out_shape=jax.ShapeDtypeStruct((125, 1, _CH), jnp.int32),
    )(x, y, z)
    return ids.reshape(_NPTS)


_GCH = 1024
_NGCH = _W // _GCH
_GSTR = _GCH // 128


def _sc_body(ids_hbm, feats_hbm, dens_hbm, dense_hbm, mask_hbm,
             lbuf, idbuf, gbuf, tbuf, mkbuf, dnbuf, semi, semg0, semg1,
             semw):
    wid = lax.axis_index("s") * _NC + lax.axis_index("c")
    lo = wid * _W
    lane = lax.iota(jnp.int32, 16)
    neg1 = jnp.full((16,), -1, jnp.int32)

    def init_body(i, carry):
        for t in range(8):
            lbuf[pl.ds(pl.multiple_of((i * 8 + t) * 16, 8), 16)] = neg1
        return carry

    lax.fori_loop(0, _W // 128, init_body, 0)

    nchunk = _NPTS // _CH

    def id_copy(k, b):
        return pltpu.make_async_copy(
            ids_hbm.at[pl.ds(pl.multiple_of(k * _CH, 8), _CH)],
            idbuf.at[b], semi)

    id_copy(0, 0).start()

    def chunk_body(k, carry):
        b = k & 1
        id_copy(k, b).wait()
        pl.when(k < nchunk - 1)(
            lambda: id_copy(k + 1, b ^ 1).start())
        base = k * _CH

        def vec_body(j, c2):
            nu = 10
            vs = [idbuf[b, pl.ds(pl.multiple_of((j * nu + t) * 16, 8),
                                 16)]
                  for t in range(nu)]
            us = [v - lo for v in vs]
            ms = [u.astype(jnp.uint32) < jnp.uint32(_W) for u in us]
            vals = [base + (j * nu + t) * 16 + lane for t in range(nu)]
            for t in range(nu):
                plsc.store_scatter(lbuf, [us[t]], vals[t], mask=ms[t])
            return c2

        lax.fori_loop(0, _CH // 160, vec_body, 0)
        return carry

    lax.fori_loop(0, nchunk, chunk_body, 0)

    def prep(c, b):
        base = c * _GCH

        def pbody(j, carry):
            for t in range(4):
                jj = j * 4 + t
                off = pl.multiple_of(base + jj * 16, 8)
                xv = lbuf[pl.ds(off, 16)]
                m = xv >= 0
                pad = (lo + base + jj * 16 + lane) & _PADMASK
                lbuf[pl.ds(off, 16)] = jnp.where(m, xv, pad)
                mkbuf[b, pl.ds(pl.multiple_of(jj * 16, 8), 16)] = (
                    jnp.where(m, 1.0, 0.0))
            return carry

        lax.fori_loop(0, _GCH // 64, pbody, 0)

    def fire_g(c, b, sg):
        for j in range(_GSTR):
            idxs = lbuf.at[pl.ds(pl.multiple_of(c * _GCH + j * 128, 8),
                                 128)]
            pltpu.make_async_copy(
                feats_hbm.at[idxs],
                gbuf.at[pl.ds(b * _GCH + j * 128, 128)], sg).start()
        pltpu.make_async_copy(
            dens_hbm.at[pl.ds(pl.multiple_of(lo + c * _GCH, 8), _GCH)],
            dnbuf.at[b], sg).start()

    def drain_g(b, sg):
        for j in range(_GSTR):
            idxs = lbuf.at[pl.ds(j * 128, 128)]
            pltpu.make_async_copy(
                feats_hbm.at[idxs],
                gbuf.at[pl.ds(b * _GCH + j * 128, 128)], sg).wait()
        pltpu.make_async_copy(
            dens_hbm.at[pl.ds(0, _GCH)], dnbuf.at[b], sg).wait()

    def xpose(b):
        def jbody(j, carry):
            moff = pl.multiple_of(j * 16, 8)
            mf = mkbuf[b, pl.ds(moff, 16)]
            dn = dnbuf[b, pl.ds(moff, 16)]
            bgz = dn * (1.0 - mf)
            rowv = b * _GCH + j * 16 + lane
            cvs = [jnp.full((16,), ch, jnp.int32) for ch in range(_C)]
            vals = [plsc.load_gather(gbuf, [rowv, cvs[ch]])
                    for ch in range(_C)]
            outs = [vals[ch] * mf for ch in range(_C)]
            outs[0] = outs[0] + bgz
            for ch in range(_C):
                tbuf[ch, pl.ds(moff, 16)] = outs[ch]
            return carry

        lax.fori_loop(0, _GCH // 16, jbody, 0)

    def fire_w(c, b):
        dst = pl.ds(pl.multiple_of(lo + c * _GCH, 8), _GCH)
        for ch in range(_C):
            pltpu.make_async_copy(
                tbuf.at[ch], dense_hbm.at[ch, dst], semw).start()
        pltpu.make_async_copy(mkbuf.at[b], mask_hbm.at[dst],
                              semw).start()

    def drain_w():
        dst = pl.ds(0, _GCH)
        for ch in range(_C):
            pltpu.make_async_copy(
                tbuf.at[ch], dense_hbm.at[ch, dst], semw).wait()
        pltpu.make_async_copy(mkbuf.at[0], mask_hbm.at[dst],
                              semw).wait()

    prep(0, 0)
    fire_g(0, 0, semg0)

    def pair_body(p, carry):
        for b in (0, 1):
            c = p * 2 + b
            sg_cur = semg0 if b == 0 else semg1
            sg_nxt = semg1 if b == 0 else semg0

            def donext(c=c, b=b, sg=sg_nxt):
                prep(c + 1, b ^ 1)
                fire_g(c + 1, b ^ 1, sg)

            pl.when(c >= 1)(drain_w)
            pl.when(c < _NGCH - 1)(donext)
            drain_g(b, sg_cur)
            xpose(b)
            fire_w(c, b)
        return carry

    lax.fori_loop(0, _NGCH // 2, pair_body, 0)
    drain_w()


@functools.cache
def _sc_main():
    return pl.kernel(
        _sc_body,
        out_type=(
            jax.ShapeDtypeStruct((_C, _NV), jnp.float32),
            jax.ShapeDtypeStruct((_NV,), jnp.float32),
        ),
        mesh=plsc.VectorSubcoreMesh(core_axis_name="c",
                                    subcore_axis_name="s",
                                    num_cores=_NC, num_subcores=_NS),
        compiler_params=pltpu.CompilerParams(needs_layout_passes=False,
                                             use_tc_tiling_on_sc=False),
        scratch_types=[
            pltpu.VMEM((_W,), jnp.int32),
            pltpu.VMEM((2, _CH), jnp.int32),
            pltpu.VMEM((2 * _GCH, _C), jnp.float32),
            pltpu.VMEM((_C, _GCH), jnp.float32),
            pltpu.VMEM((2, _GCH), jnp.float32),
            pltpu.VMEM((2, _GCH), jnp.float32),
            pltpu.SemaphoreType.DMA,
            pltpu.SemaphoreType.DMA,
            pltpu.SemaphoreType.DMA,
            pltpu.SemaphoreType.DMA,
        ],
    )


def _upsample_matrix():
    xs = np.clip((np.arange(128, dtype=np.float64) + 0.5) / 2.0 - 0.5,
                 0.0, 63.0)
    x0 = np.floor(xs).astype(np.int64)
    x1 = np.minimum(x0 + 1, 63)
    w = xs - x0
    u = np.zeros((128, 64), dtype=np.float32)
    u[np.arange(128), x0] += (1.0 - w).astype(np.float32)
    u[np.arange(128), x1] += w.astype(np.float32)
    return u


def _ups_body(pre_ref, uy_ref, uyt_ref, o_ref):
    px = pl.program_id(0)
    xf = jnp.clip((px.astype(jnp.float32) + 0.5) / 2.0 - 0.5, 0.0, 63.0)
    x0 = jnp.floor(xf).astype(jnp.int32)
    x1 = jnp.minimum(x0 + 1, 63)
    w = xf - x0.astype(jnp.float32)
    pa = pre_ref[pl.ds(x0, 1)][0]
    pb = pre_ref[pl.ds(x1, 1)][0]
    plane = pa * (1.0 - w) + pb * w
    d = jnp.dot(jnp.dot(uy_ref[...], plane,
                        preferred_element_type=jnp.float32),
                uyt_ref[...],
                preferred_element_type=jnp.float32)
    o_ref[0, 0, :] = d.reshape(_VOL * _VOL)


def _upsample(pre, uy, uyt):
    dens = pl.pallas_call(
        _ups_body,
        grid=(_VOL,),
        in_specs=[
            pl.BlockSpec((64, 64, 64), lambda i: (0, 0, 0)),
            pl.BlockSpec((128, 64), lambda i: (0, 0)),
            pl.BlockSpec((64, 128), lambda i: (0, 0)),
        ],
        out_specs=pl.BlockSpec((1, 1, _VOL * _VOL), lambda i: (i, 0, 0)),
        out_shape=jax.ShapeDtypeStruct((_VOL, 1, _VOL * _VOL),
                                       jnp.float32),
    )(pre, uy, uyt)
    return dens.reshape(_NV)


def kernel(feats, coords, pre_volume):
    ids = _flat_ids(coords.astype(jnp.int32))
    uy = jnp.asarray(_upsample_matrix())
    dens = _upsample(pre_volume.reshape(64, 64, 64), uy, uy.T)
    dense, mask = _sc_main()(ids, feats, dens)
    dense_volume = dense.reshape(1, _C, _VOL, _VOL, _VOL)
    mask_volume = mask.reshape(1, 1, _VOL, _VOL, _VOL)
    return dense_volume, mask_volume


# speedup vs baseline: 18.3050x; 1.2037x over previous
import functools

import numpy as np
import jax
import jax.numpy as jnp
from jax import lax
from jax.experimental import pallas as pl
from jax.experimental.pallas import tpu as pltpu
from jax.experimental.pallas import tpu_sc as plsc

_VOL = 128
_NV = _VOL * _VOL * _VOL
_C = 16
_NPTS = 500000
_NC = 2
_NS = 16
_NW = _NC * _NS
_W = _NV // _NW
_WROWS = _W // 128
_CH = 4000
_PADMASK = (1 << 18) - 1


def _ids_body(x_ref, y_ref, z_ref, o_ref):
    o_ref[...] = (x_ref[...] * (_VOL * _VOL) + y_ref[...] * _VOL
                  + z_ref[...])


def _flat_ids(coords):
    nb, bs = 8, _NPTS // 8
    x = coords[:, 0].reshape(nb, 1, bs)
    y = coords[:, 1].reshape(nb, 1, bs)
    z = coords[:, 2].reshape(nb, 1, bs)
    spec = pl.BlockSpec((1, 1, bs), lambda i: (i, 0, 0))
    ids = pl.pallas_call(
        _ids_body,
        grid=(nb,),
        in_specs=[spec, spec, spec],
        out_specs=spec,
        out_shape=jax.ShapeDtypeStruct((nb, 1, bs), jnp.int32),
    )(x, y, z)
    return ids.reshape(_NPTS)


_GCH = 1024
_NGCH = _W // _GCH
_GSTR = _GCH // 128


def _sc_body(ids_hbm, feats_hbm, dens_hbm, dense_hbm, mask_hbm,
             lbuf, idbuf, gbuf, tbuf, mkbuf, dnbuf, semi, semg0, semg1,
             semw):
    wid = lax.axis_index("s") * _NC + lax.axis_index("c")
    lo = wid * _W
    lane = lax.iota(jnp.int32, 16)
    neg1 = jnp.full((16,), -1, jnp.int32)

    def init_body(i, carry):
        for t in range(8):
            lbuf[pl.ds(pl.multiple_of((i * 8 + t) * 16, 8), 16)] = neg1
        return carry

    lax.fori_loop(0, _W // 128, init_body, 0)

    nchunk = _NPTS // _CH

    def id_copy(k, b):
        return pltpu.make_async_copy(
            ids_hbm.at[pl.ds(pl.multiple_of(k * _CH, 8), _CH)],
            idbuf.at[b], semi)

    id_copy(0, 0).start()

    def chunk_body(k, carry):
        b = k & 1
        id_copy(k, b).wait()
        pl.when(k < nchunk - 1)(
            lambda: id_copy(k + 1, b ^ 1).start())
        base = k * _CH

        def vec_body(j, c2):
            nu = 10
            vs = [idbuf[b, pl.ds(pl.multiple_of((j * nu + t) * 16, 8),
                                 16)]
                  for t in range(nu)]
            us = [v - lo for v in vs]
            ms = [u.astype(jnp.uint32) < jnp.uint32(_W) for u in us]
            vals = [base + (j * nu + t) * 16 + lane for t in range(nu)]
            for t in range(nu):
                plsc.store_scatter(lbuf, [us[t]], vals[t], mask=ms[t])
            return c2

        lax.fori_loop(0, _CH // 160, vec_body, 0)
        return carry

    lax.fori_loop(0, nchunk, chunk_body, 0)

    def prep(c, b):
        base = c * _GCH

        def pbody(j, carry):
            nu = 8
            offs = [pl.multiple_of(base + (j * nu + t) * 16, 8)
                    for t in range(nu)]
            moffs = [pl.multiple_of((j * nu + t) * 16, 8)
                     for t in range(nu)]
            xs = [lbuf[pl.ds(o, 16)] for o in offs]
            ms = [x >= 0 for x in xs]
            pads = [(lo + base + (j * nu + t) * 16 + lane) & _PADMASK
                    for t in range(nu)]
            cl = [jnp.where(ms[t], xs[t], pads[t]) for t in range(nu)]
            mk = [jnp.where(ms[t], 1.0, 0.0) for t in range(nu)]
            for t in range(nu):
                lbuf[pl.ds(offs[t], 16)] = cl[t]
                mkbuf[b, pl.ds(moffs[t], 16)] = mk[t]
            return carry

        lax.fori_loop(0, _GCH // 128, pbody, 0)

    def fire_g(c, b, sg):
        for j in range(_GSTR):
            idxs = lbuf.at[pl.ds(pl.multiple_of(c * _GCH + j * 128, 8),
                                 128)]
            pltpu.make_async_copy(
                feats_hbm.at[idxs],
                gbuf.at[pl.ds(b * _GCH + j * 128, 128)], sg).start()
        pltpu.make_async_copy(
            dens_hbm.at[pl.ds(pl.multiple_of(lo + c * _GCH, 8), _GCH)],
            dnbuf.at[b], sg).start()

    def drain_g(b, sg):
        for j in range(_GSTR):
            idxs = lbuf.at[pl.ds(j * 128, 128)]
            pltpu.make_async_copy(
                feats_hbm.at[idxs],
                gbuf.at[pl.ds(b * _GCH + j * 128, 128)], sg).wait()
        pltpu.make_async_copy(
            dens_hbm.at[pl.ds(0, _GCH)], dnbuf.at[b], sg).wait()

    def xpose(b):
        def jbody(j, carry):
            moff = pl.multiple_of(j * 16, 8)
            mf = mkbuf[b, pl.ds(moff, 16)]
            dn = dnbuf[b, pl.ds(moff, 16)]
            bgz = dn * (1.0 - mf)
            rowv = b * _GCH + j * 16 + lane
            cvs = [jnp.full((16,), ch, jnp.int32) for ch in range(_C)]
            vals = [plsc.load_gather(gbuf, [rowv, cvs[ch]])
                    for ch in range(_C)]
            outs = [vals[ch] * mf for ch in range(_C)]
            outs[0] = outs[0] + bgz
            for ch in range(_C):
                tbuf[ch, pl.ds(moff, 16)] = outs[ch]
            return carry

        lax.fori_loop(0, _GCH // 16, jbody, 0)

    def fire_w(c, b):
        dst = pl.ds(pl.multiple_of(lo + c * _GCH, 8), _GCH)
        for ch in range(_C):
            pltpu.make_async_copy(
                tbuf.at[ch], dense_hbm.at[ch, dst], semw).start()
        pltpu.make_async_copy(mkbuf.at[b], mask_hbm.at[dst],
                              semw).start()

    def drain_w():
        dst = pl.ds(0, _GCH)
        for ch in range(_C):
            pltpu.make_async_copy(
                tbuf.at[ch], dense_hbm.at[ch, dst], semw).wait()
        pltpu.make_async_copy(mkbuf.at[0], mask_hbm.at[dst],
                              semw).wait()

    prep(0, 0)
    fire_g(0, 0, semg0)

    def pair_body(p, carry):
        for b in (0, 1):
            c = p * 2 + b
            sg_cur = semg0 if b == 0 else semg1
            sg_nxt = semg1 if b == 0 else semg0

            def donext(c=c, b=b, sg=sg_nxt):
                prep(c + 1, b ^ 1)
                fire_g(c + 1, b ^ 1, sg)

            pl.when(c >= 1)(drain_w)
            pl.when(c < _NGCH - 1)(donext)
            drain_g(b, sg_cur)
            xpose(b)
            fire_w(c, b)
        return carry

    lax.fori_loop(0, _NGCH // 2, pair_body, 0)
    drain_w()


@functools.cache
def _sc_main():
    return pl.kernel(
        _sc_body,
        out_type=(
            jax.ShapeDtypeStruct((_C, _NV), jnp.float32),
            jax.ShapeDtypeStruct((_NV,), jnp.float32),
        ),
        mesh=plsc.VectorSubcoreMesh(core_axis_name="c",
                                    subcore_axis_name="s",
                                    num_cores=_NC, num_subcores=_NS),
        compiler_params=pltpu.CompilerParams(needs_layout_passes=False,
                                             use_tc_tiling_on_sc=False),
        scratch_types=[
            pltpu.VMEM((_W,), jnp.int32),
            pltpu.VMEM((2, _CH), jnp.int32),
            pltpu.VMEM((2 * _GCH, _C), jnp.float32),
            pltpu.VMEM((_C, _GCH), jnp.float32),
            pltpu.VMEM((2, _GCH), jnp.float32),
            pltpu.VMEM((2, _GCH), jnp.float32),
            pltpu.SemaphoreType.DMA,
            pltpu.SemaphoreType.DMA,
            pltpu.SemaphoreType.DMA,
            pltpu.SemaphoreType.DMA,
        ],
    )


def _upsample_matrix():
    xs = np.clip((np.arange(128, dtype=np.float64) + 0.5) / 2.0 - 0.5,
                 0.0, 63.0)
    x0 = np.floor(xs).astype(np.int64)
    x1 = np.minimum(x0 + 1, 63)
    w = xs - x0
    u = np.zeros((128, 64), dtype=np.float32)
    u[np.arange(128), x0] += (1.0 - w).astype(np.float32)
    u[np.arange(128), x1] += w.astype(np.float32)
    return u


_UPP = 8


def _ups_body(pre_ref, uy_ref, uyt_ref, o_ref):
    pid = pl.program_id(0)
    for t in range(_UPP):
        px = pid * _UPP + t
        xf = jnp.clip((px.astype(jnp.float32) + 0.5) / 2.0 - 0.5,
                      0.0, 63.0)
        x0 = jnp.floor(xf).astype(jnp.int32)
        x1 = jnp.minimum(x0 + 1, 63)
        w = xf - x0.astype(jnp.float32)
        pa = pre_ref[pl.ds(x0, 1)][0]
        pb = pre_ref[pl.ds(x1, 1)][0]
        plane = pa * (1.0 - w) + pb * w
        d = jnp.dot(jnp.dot(uy_ref[...], plane,
                            preferred_element_type=jnp.float32),
                    uyt_ref[...],
                    preferred_element_type=jnp.float32)
        o_ref[t, 0, :] = d.reshape(_VOL * _VOL)


def _upsample(pre, uy, uyt):
    dens = pl.pallas_call(
        _ups_body,
        grid=(_VOL // _UPP,),
        in_specs=[
            pl.BlockSpec((64, 64, 64), lambda i: (0, 0, 0)),
            pl.BlockSpec((128, 64), lambda i: (0, 0)),
            pl.BlockSpec((64, 128), lambda i: (0, 0)),
        ],
        out_specs=pl.BlockSpec((_UPP, 1, _VOL * _VOL),
                               lambda i: (i, 0, 0)),
        out_shape=jax.ShapeDtypeStruct((_VOL, 1, _VOL * _VOL),
                                       jnp.float32),
    )(pre, uy, uyt)
    return dens.reshape(_NV)


def kernel(feats, coords, pre_volume):
    ids = _flat_ids(coords.astype(jnp.int32))
    uy = jnp.asarray(_upsample_matrix())
    dens = _upsample(pre_volume.reshape(64, 64, 64), uy, uy.T)
    dense, mask = _sc_main()(ids, feats, dens)
    dense_volume = dense.reshape(1, _C, _VOL, _VOL, _VOL)
    mask_volume = mask.reshape(1, 1, _VOL, _VOL, _VOL)
    return dense_volume, mask_volume


# speedup vs baseline: 18.3123x; 1.0004x over previous
import functools

import numpy as np
import jax
import jax.numpy as jnp
from jax import lax
from jax.experimental import pallas as pl
from jax.experimental.pallas import tpu as pltpu
from jax.experimental.pallas import tpu_sc as plsc

_VOL = 128
_NV = _VOL * _VOL * _VOL
_C = 16
_NPTS = 500000
_NC = 2
_NS = 16
_NW = _NC * _NS
_W = _NV // _NW
_WROWS = _W // 128
_CH = 4000
_PADMASK = (1 << 18) - 1


def _ids_body(x_ref, y_ref, z_ref, o_ref):
    o_ref[...] = (x_ref[...] * (_VOL * _VOL) + y_ref[...] * _VOL
                  + z_ref[...])


def _flat_ids(coords):
    nb, bs = 8, _NPTS // 8
    x = coords[:, 0].reshape(nb, 1, bs)
    y = coords[:, 1].reshape(nb, 1, bs)
    z = coords[:, 2].reshape(nb, 1, bs)
    spec = pl.BlockSpec((1, 1, bs), lambda i: (i, 0, 0))
    ids = pl.pallas_call(
        _ids_body,
        grid=(nb,),
        in_specs=[spec, spec, spec],
        out_specs=spec,
        out_shape=jax.ShapeDtypeStruct((nb, 1, bs), jnp.int32),
    )(x, y, z)
    return ids.reshape(_NPTS)


_GCH = 1024
_NGCH = _W // _GCH
_GSTR = _GCH // 128


def _sc_body(ids_hbm, feats_hbm, dens_hbm, dense_hbm, mask_hbm,
             lbuf, idbuf, gbuf, tbuf, mkbuf, dnbuf, semi, semg0, semg1,
             semw):
    wid = lax.axis_index("s") * _NC + lax.axis_index("c")
    lo = wid * _W
    lane = lax.iota(jnp.int32, 16)
    neg1 = jnp.full((16,), -1, jnp.int32)

    def init_body(i, carry):
        for t in range(8):
            lbuf[pl.ds(pl.multiple_of((i * 8 + t) * 16, 8), 16)] = neg1
        return carry

    lax.fori_loop(0, _W // 128, init_body, 0)

    nchunk = _NPTS // _CH

    def id_copy(k, b):
        return pltpu.make_async_copy(
            ids_hbm.at[pl.ds(pl.multiple_of(k * _CH, 8), _CH)],
            idbuf.at[b], semi)

    id_copy(0, 0).start()

    def chunk_body(k, carry):
        b = k & 1
        id_copy(k, b).wait()
        pl.when(k < nchunk - 1)(
            lambda: id_copy(k + 1, b ^ 1).start())
        base = k * _CH

        def vec_body(j, c2):
            nu = 10
            vs = [idbuf[b, pl.ds(pl.multiple_of((j * nu + t) * 16, 8),
                                 16)]
                  for t in range(nu)]
            us = [v - lo for v in vs]
            ms = [u.astype(jnp.uint32) < jnp.uint32(_W) for u in us]
            vals = [base + (j * nu + t) * 16 + lane for t in range(nu)]
            for t in range(nu):
                plsc.store_scatter(lbuf, [us[t]], vals[t], mask=ms[t])
            return c2

        lax.fori_loop(0, _CH // 160, vec_body, 0)
        return carry

    lax.fori_loop(0, nchunk, chunk_body, 0)

    def prep(c, b):
        base = c * _GCH

        def pbody(j, carry):
            nu = 8
            offs = [pl.multiple_of(base + (j * nu + t) * 16, 8)
                    for t in range(nu)]
            moffs = [pl.multiple_of((j * nu + t) * 16, 8)
                     for t in range(nu)]
            xs = [lbuf[pl.ds(o, 16)] for o in offs]
            ms = [x >= 0 for x in xs]
            pads = [(lo + base + (j * nu + t) * 16 + lane) & _PADMASK
                    for t in range(nu)]
            cl = [jnp.where(ms[t], xs[t], pads[t]) for t in range(nu)]
            mk = [jnp.where(ms[t], 1.0, 0.0) for t in range(nu)]
            for t in range(nu):
                lbuf[pl.ds(offs[t], 16)] = cl[t]
                mkbuf[b, pl.ds(moffs[t], 16)] = mk[t]
            return carry

        lax.fori_loop(0, _GCH // 128, pbody, 0)

    def fire_g(c, b, sg):
        for j in range(_GSTR):
            idxs = lbuf.at[pl.ds(pl.multiple_of(c * _GCH + j * 128, 8),
                                 128)]
            pltpu.make_async_copy(
                feats_hbm.at[idxs],
                gbuf.at[pl.ds(b * _GCH + j * 128, 128)], sg).start()
        pltpu.make_async_copy(
            dens_hbm.at[pl.ds(pl.multiple_of(lo + c * _GCH, 8), _GCH)],
            dnbuf.at[b], sg).start()

    def drain_g(b, sg):
        for j in range(_GSTR):
            idxs = lbuf.at[pl.ds(j * 128, 128)]
            pltpu.make_async_copy(
                feats_hbm.at[idxs],
                gbuf.at[pl.ds(b * _GCH + j * 128, 128)], sg).wait()
        pltpu.make_async_copy(
            dens_hbm.at[pl.ds(0, _GCH)], dnbuf.at[b], sg).wait()

    def xpose(b):
        def jbody(j, carry):
            moff = pl.multiple_of(j * 16, 8)
            mf = mkbuf[b, pl.ds(moff, 16)]
            dn = dnbuf[b, pl.ds(moff, 16)]
            bgz = dn * (1.0 - mf)
            rowv = b * _GCH + j * 16 + lane
            cvs = [jnp.full((16,), ch, jnp.int32) for ch in range(_C)]
            vals = [plsc.load_gather(gbuf, [rowv, cvs[ch]])
                    for ch in range(_C)]
            outs = [vals[ch] * mf for ch in range(_C)]
            outs[0] = outs[0] + bgz
            for ch in range(_C):
                tbuf[ch, pl.ds(moff, 16)] = outs[ch]
            return carry

        lax.fori_loop(0, _GCH // 16, jbody, 0)

    def fire_w(c, b):
        base = lo + c * _GCH
        for ch in range(_C):
            dst = pl.ds(pl.multiple_of(ch * _NV + base, 8), _GCH)
            pltpu.make_async_copy(tbuf.at[ch], dense_hbm.at[dst],
                                  semw).start()
        pltpu.make_async_copy(
            mkbuf.at[b],
            mask_hbm.at[pl.ds(pl.multiple_of(base, 8), _GCH)],
            semw).start()

    def drain_w():
        dst = pl.ds(0, _GCH)
        for ch in range(_C):
            pltpu.make_async_copy(tbuf.at[ch], dense_hbm.at[dst],
                                  semw).wait()
        pltpu.make_async_copy(mkbuf.at[0], mask_hbm.at[dst],
                              semw).wait()

    prep(0, 0)
    fire_g(0, 0, semg0)

    def pair_body(p, carry):
        for b in (0, 1):
            c = p * 2 + b
            sg_cur = semg0 if b == 0 else semg1
            sg_nxt = semg1 if b == 0 else semg0

            def donext(c=c, b=b, sg=sg_nxt):
                prep(c + 1, b ^ 1)
                fire_g(c + 1, b ^ 1, sg)

            pl.when(c >= 1)(drain_w)
            pl.when(c < _NGCH - 1)(donext)
            drain_g(b, sg_cur)
            xpose(b)
            fire_w(c, b)
        return carry

    lax.fori_loop(0, _NGCH // 2, pair_body, 0)
    drain_w()


@functools.cache
def _sc_main():
    return pl.kernel(
        _sc_body,
        out_type=(
            jax.ShapeDtypeStruct((_C * _NV,), jnp.float32),
            jax.ShapeDtypeStruct((_NV,), jnp.float32),
        ),
        mesh=plsc.VectorSubcoreMesh(core_axis_name="c",
                                    subcore_axis_name="s",
                                    num_cores=_NC, num_subcores=_NS),
        compiler_params=pltpu.CompilerParams(needs_layout_passes=False,
                                             use_tc_tiling_on_sc=False),
        scratch_types=[
            pltpu.VMEM((_W,), jnp.int32),
            pltpu.VMEM((2, _CH), jnp.int32),
            pltpu.VMEM((2 * _GCH, _C), jnp.float32),
            pltpu.VMEM((_C, _GCH), jnp.float32),
            pltpu.VMEM((2, _GCH), jnp.float32),
            pltpu.VMEM((2, _GCH), jnp.float32),
            pltpu.SemaphoreType.DMA,
            pltpu.SemaphoreType.DMA,
            pltpu.SemaphoreType.DMA,
            pltpu.SemaphoreType.DMA,
        ],
    )


def _upsample_matrix():
    xs = np.clip((np.arange(128, dtype=np.float64) + 0.5) / 2.0 - 0.5,
                 0.0, 63.0)
    x0 = np.floor(xs).astype(np.int64)
    x1 = np.minimum(x0 + 1, 63)
    w = xs - x0
    u = np.zeros((128, 64), dtype=np.float32)
    u[np.arange(128), x0] += (1.0 - w).astype(np.float32)
    u[np.arange(128), x1] += w.astype(np.float32)
    return u


_UPP = 8


def _ups_body(pre_ref, uy_ref, uyt_ref, o_ref):
    pid = pl.program_id(0)
    for t in range(_UPP):
        px = pid * _UPP + t
        xf = jnp.clip((px.astype(jnp.float32) + 0.5) / 2.0 - 0.5,
                      0.0, 63.0)
        x0 = jnp.floor(xf).astype(jnp.int32)
        x1 = jnp.minimum(x0 + 1, 63)
        w = xf - x0.astype(jnp.float32)
        pa = pre_ref[pl.ds(x0, 1)][0]
        pb = pre_ref[pl.ds(x1, 1)][0]
        plane = pa * (1.0 - w) + pb * w
        d = jnp.dot(jnp.dot(uy_ref[...], plane,
                            preferred_element_type=jnp.float32),
                    uyt_ref[...],
                    preferred_element_type=jnp.float32)
        o_ref[t, 0, :] = d.reshape(_VOL * _VOL)


def _upsample(pre, uy, uyt):
    dens = pl.pallas_call(
        _ups_body,
        grid=(_VOL // _UPP,),
        in_specs=[
            pl.BlockSpec((64, 64, 64), lambda i: (0, 0, 0)),
            pl.BlockSpec((128, 64), lambda i: (0, 0)),
            pl.BlockSpec((64, 128), lambda i: (0, 0)),
        ],
        out_specs=pl.BlockSpec((_UPP, 1, _VOL * _VOL),
                               lambda i: (i, 0, 0)),
        out_shape=jax.ShapeDtypeStruct((_VOL, 1, _VOL * _VOL),
                                       jnp.float32),
    )(pre, uy, uyt)
    return dens.reshape(_NV)


def kernel(feats, coords, pre_volume):
    ids = _flat_ids(coords.astype(jnp.int32))
    uy = jnp.asarray(_upsample_matrix())
    dens = _upsample(pre_volume.reshape(64, 64, 64), uy, uy.T)
    dense, mask = _sc_main()(ids, feats, dens)
    dense_volume = dense.reshape(1, _C, _VOL, _VOL, _VOL)
    mask_volume = mask.reshape(1, 1, _VOL, _VOL, _VOL)
    return dense_volume, mask_volume
